# Initial kernel scaffold; baseline (speedup 1.0000x reference)
import functools

import jax
import jax.numpy as jnp
from jax import lax
from jax.experimental import pallas as pl
from jax.experimental.pallas import tpu as pltpu
from jax.experimental.pallas import tpu_sc as plsc

NC = 2
NS = 16
NW = NC * NS
CHUNK = 128
NBUF = 4


def _sc_gather(in_emb, out_emb, cen_idx, cat_idx, B, D, KP1):
    cen_w = B // NW // CHUNK
    cat_w = B * KP1 // NW // CHUNK
    mesh = plsc.VectorSubcoreMesh(core_axis_name="c", subcore_axis_name="s")

    @functools.partial(
        pl.kernel,
        mesh=mesh,
        out_type=[
            jax.ShapeDtypeStruct((B, D), jnp.float32),
            jax.ShapeDtypeStruct((B * KP1, D), jnp.float32),
        ],
        scratch_types=[
            pltpu.VMEM((cen_w, CHUNK), jnp.int32),
            pltpu.VMEM((cat_w, CHUNK), jnp.int32),
            pltpu.VMEM((NBUF, CHUNK, D), jnp.float32),
            pltpu.SemaphoreType.DMA((NBUF,)),
        ],
    )
    def k(in_hbm, out_hbm, cenidx_hbm, catidx_hbm, v_out, cat_out,
          cen_v, cat_v, bufs, sems):
        w = lax.axis_index("s") * NC + lax.axis_index("c")
        pltpu.sync_copy(cenidx_hbm.at[pl.ds(w * cen_w, cen_w)], cen_v)
        pltpu.sync_copy(catidx_hbm.at[pl.ds(w * cat_w, cat_w)], cat_v)

        vbase = w * (B // NW)
        cps = [
            pltpu.async_copy(in_hbm.at[cen_v.at[j]], bufs.at[j], sems.at[j])
            for j in range(cen_w)
        ]
        for j in range(cen_w):
            cps[j].wait()
            pltpu.sync_copy(bufs.at[j], v_out.at[pl.ds(vbase + j * CHUNK, CHUNK)])

        cbase = w * (B * KP1 // NW)
        for j in range(NBUF):
            pltpu.async_copy(out_hbm.at[cat_v.at[j]], bufs.at[j], sems.at[j])

        steady = cat_w - NBUF

        @pl.loop(0, steady, step=NBUF)
        def _(c):
            for j in range(NBUF):
                pltpu.make_async_copy(
                    out_hbm.at[cat_v.at[j]], bufs.at[j], sems.at[j]
                ).wait()
                pltpu.sync_copy(
                    bufs.at[j], cat_out.at[pl.ds(cbase + (c + j) * CHUNK, CHUNK)]
                )
                pltpu.async_copy(
                    out_hbm.at[cat_v.at[c + j + NBUF]], bufs.at[j], sems.at[j]
                )

        for j in range(NBUF):
            pltpu.make_async_copy(
                out_hbm.at[cat_v.at[j]], bufs.at[j], sems.at[j]
            ).wait()
            pltpu.sync_copy(
                bufs.at[j], cat_out.at[pl.ds(cbase + (steady + j) * CHUNK, CHUNK)]
            )

    return k(in_emb, out_emb, cen_idx, cat_idx)


def _softplus(x):
    return jnp.maximum(x, 0.0) + jnp.log1p(jnp.exp(-jnp.abs(x)))


def _tc_loss(v_rows, cat_rows, B, KP1, D):
    BB = 512
    grid = B // BB
    inv_b = 1.0 / B

    def body(v_ref, cat_ref, o_ref):
        i = pl.program_id(0)
        v = v_ref[...]
        cat = cat_ref[...]
        logits = jnp.sum(v[:, None, :] * cat, axis=-1)
        pos = logits[:, 0]
        neg = logits[:, 1:]
        blk = jnp.sum(_softplus(-pos)) + jnp.sum(_softplus(neg))

        @pl.when(i == 0)
        def _():
            o_ref[0] = 0.0

        o_ref[0] += blk * inv_b

    return pl.pallas_call(
        body,
        grid=(grid,),
        in_specs=[
            pl.BlockSpec((BB, D), lambda i: (i, 0)),
            pl.BlockSpec((BB, KP1, D), lambda i: (i, 0, 0)),
        ],
        out_specs=pl.BlockSpec(
            (1,), lambda i: (0,), memory_space=pltpu.SMEM
        ),
        out_shape=jax.ShapeDtypeStruct((1,), jnp.float32),
    )(v_rows, cat_rows)


def kernel(center, pos, neg, in_emb, out_emb):
    B = center.shape[0]
    K = neg.shape[1]
    D = in_emb.shape[1]
    KP1 = K + 1

    cen_idx = center.astype(jnp.int32).reshape(B // CHUNK, CHUNK)
    cat_idx = (
        jnp.concatenate([pos.astype(jnp.int32)[:, None], neg.astype(jnp.int32)], axis=1)
        .reshape(B * KP1 // CHUNK, CHUNK)
    )

    v_rows, cat_rows = _sc_gather(in_emb, out_emb, cen_idx, cat_idx, B, D, KP1)
    loss = _tc_loss(v_rows, cat_rows.reshape(B, KP1, D), B, KP1, D)
    return loss


# speedup vs baseline: 2.8072x; 2.8072x over previous
import functools

import jax
import jax.numpy as jnp
from jax import lax
from jax.experimental import pallas as pl
from jax.experimental.pallas import tpu as pltpu
from jax.experimental.pallas import tpu_sc as plsc

NC = 2
NS = 16
NW = NC * NS
CHUNK = 128
NBUF = 4


def _sc_gather(in_emb, out_emb, cen_idx, cat_idx, B, D, KP1):
    cen_w = B // NW // CHUNK
    cat_w = B * KP1 // NW // CHUNK
    mesh = plsc.VectorSubcoreMesh(core_axis_name="c", subcore_axis_name="s")

    @functools.partial(
        pl.kernel,
        mesh=mesh,
        compiler_params=pltpu.CompilerParams(use_tc_tiling_on_sc=False),
        out_type=[
            jax.ShapeDtypeStruct((B, D), jnp.float32),
            jax.ShapeDtypeStruct((B * KP1, D), jnp.float32),
        ],
        scratch_types=[
            pltpu.VMEM((cen_w, CHUNK), jnp.int32),
            pltpu.VMEM((cat_w, CHUNK), jnp.int32),
            pltpu.VMEM((NBUF, CHUNK, D), jnp.float32),
            pltpu.SemaphoreType.DMA((NBUF,)),
        ],
    )
    def k(in_hbm, out_hbm, cenidx_hbm, catidx_hbm, v_out, cat_out,
          cen_v, cat_v, bufs, sems):
        w = lax.axis_index("s") * NC + lax.axis_index("c")
        pltpu.sync_copy(cenidx_hbm.at[w], cen_v)
        pltpu.sync_copy(catidx_hbm.at[w], cat_v)

        vbase = w * (B // NW)
        cps = [
            pltpu.async_copy(in_hbm.at[cen_v.at[j]], bufs.at[j], sems.at[j])
            for j in range(cen_w)
        ]
        for j in range(cen_w):
            cps[j].wait()
            pltpu.sync_copy(bufs.at[j], v_out.at[pl.ds(vbase + j * CHUNK, CHUNK)])

        cbase = w * (B * KP1 // NW)
        for j in range(NBUF):
            pltpu.async_copy(out_hbm.at[cat_v.at[j]], bufs.at[j], sems.at[j])

        steady = cat_w - NBUF

        @pl.loop(0, steady, step=NBUF)
        def _(c):
            for j in range(NBUF):
                pltpu.make_async_copy(
                    out_hbm.at[cat_v.at[j]], bufs.at[j], sems.at[j]
                ).wait()
                pltpu.sync_copy(
                    bufs.at[j], cat_out.at[pl.ds(cbase + (c + j) * CHUNK, CHUNK)]
                )
                pltpu.async_copy(
                    out_hbm.at[cat_v.at[c + j + NBUF]], bufs.at[j], sems.at[j]
                )

        for j in range(NBUF):
            pltpu.make_async_copy(
                out_hbm.at[cat_v.at[j]], bufs.at[j], sems.at[j]
            ).wait()
            pltpu.sync_copy(
                bufs.at[j], cat_out.at[pl.ds(cbase + (steady + j) * CHUNK, CHUNK)]
            )

    return k(in_emb, out_emb, cen_idx, cat_idx)


def _softplus(x):
    return jnp.maximum(x, 0.0) + jnp.log1p(jnp.exp(-jnp.abs(x)))


def _tc_loss(v_rows, cat_mat, sel, B, KP1, D):
    BB = 1024
    grid = B // BB
    inv_b = 1.0 / B
    W = KP1 * D

    def body(v_ref, cat_ref, sel_ref, o_ref):
        i = pl.program_id(0)
        v = v_ref[...]
        cat = cat_ref[...]
        vt = jnp.concatenate([v] * KP1, axis=1)
        prod = vt * cat
        logits = jax.lax.dot_general(
            prod, sel_ref[...],
            dimension_numbers=(((1,), (0,)), ((), ())),
            preferred_element_type=jnp.float32,
        )
        col = jax.lax.broadcasted_iota(jnp.int32, logits.shape, 1)
        signed = jnp.where(col == 0, -logits, logits)
        sp = _softplus(signed)
        blk = jnp.sum(jnp.where(col < KP1, sp, 0.0))

        @pl.when(i == 0)
        def _():
            o_ref[0] = 0.0

        o_ref[0] += blk * inv_b

    return pl.pallas_call(
        body,
        grid=(grid,),
        in_specs=[
            pl.BlockSpec((BB, D), lambda i: (i, 0)),
            pl.BlockSpec((BB, W), lambda i: (i, 0)),
            pl.BlockSpec((W, KP1), lambda i: (0, 0)),
        ],
        out_specs=pl.BlockSpec(
            (1,), lambda i: (0,), memory_space=pltpu.SMEM
        ),
        out_shape=jax.ShapeDtypeStruct((1,), jnp.float32),
    )(v_rows, cat_mat, sel)


def kernel(center, pos, neg, in_emb, out_emb):
    B = center.shape[0]
    K = neg.shape[1]
    D = in_emb.shape[1]
    KP1 = K + 1

    cen_idx = center.astype(jnp.int32).reshape(NW, B // NW // CHUNK, CHUNK)
    cat_idx = (
        jnp.concatenate([pos.astype(jnp.int32)[:, None], neg.astype(jnp.int32)], axis=1)
        .reshape(NW, B * KP1 // NW // CHUNK, CHUNK)
    )
    sel = (jnp.arange(KP1 * D)[:, None] // D == jnp.arange(KP1)[None, :]).astype(
        jnp.float32
    )

    v_rows, cat_rows = _sc_gather(in_emb, out_emb, cen_idx, cat_idx, B, D, KP1)
    loss = _tc_loss(v_rows, cat_rows.reshape(B, KP1 * D), sel, B, KP1, D)
    return loss
